# Initial kernel scaffold; baseline (speedup 1.0000x reference)
import jax
import jax.numpy as jnp
from jax.experimental import pallas as pl


def kernel(x, edge_index, W1, b1, W2, b2):
    raise NotImplementedError("write your pallas kernel here")


# speedup vs baseline: 12.9718x; 12.9718x over previous
import functools

import jax
import jax.numpy as jnp
from jax import lax
from jax.experimental import pallas as pl
from jax.experimental.pallas import tpu as pltpu
from jax.experimental.pallas import tpu_sc as plsc

N = 10000
NP = 10240
E = 160000
EP = 163840
D = 256
DH = 128
D2 = 128
RB = 2048
TPR = NP // 16
C1 = 80
C2 = 40


def _mesh():
    return plsc.VectorSubcoreMesh(core_axis_name="c", subcore_axis_name="s")


@functools.partial(
    pl.kernel,
    out_type=jax.ShapeDtypeStruct((2 * NP, 128), jnp.float32),
    mesh=_mesh(),
    scratch_types=[
        pltpu.VMEM_SHARED((NP, 128), jnp.float32),
        pltpu.VMEM((C2, 128), jnp.int32),
        pltpu.VMEM((128, 128), jnp.float32),
    ],
)
def _deg_kernel(dst_hbm, ones_hbm, zeros_hbm, out_hbm, acc, dstv, onesv):
    c = lax.axis_index("c")
    s = lax.axis_index("s")
    r0 = s * TPR
    pltpu.sync_copy(zeros_hbm, acc.at[pl.ds(r0, TPR)])
    pltpu.sync_copy(ones_hbm, onesv)
    pltpu.sync_copy(dst_hbm.at[pl.ds((c * 16 + s) * C2, C2)], dstv)
    plsc.subcore_barrier()

    def body(i, carry):
        pltpu.sync_copy(onesv, acc.at[dstv.at[i]], add=True)
        return carry

    lax.fori_loop(0, C2, body, 0)
    plsc.subcore_barrier()
    pltpu.sync_copy(acc.at[pl.ds(r0, TPR)], out_hbm.at[pl.ds(c * NP + r0, TPR)])


@functools.partial(
    pl.kernel,
    out_type=jax.ShapeDtypeStruct((2 * NP, DH), jnp.float32),
    mesh=_mesh(),
    scratch_types=[
        pltpu.VMEM_SHARED((NP, DH), jnp.float32),
        pltpu.VMEM((C1, 128), jnp.int32),
        pltpu.VMEM((C1, 128), jnp.int32),
        pltpu.VMEM((128, DH), jnp.float32),
        pltpu.SemaphoreType.DMA,
    ],
)
def _agg1_kernel(tab_hbm, srcb_hbm, dst_hbm, zeros_hbm, out_hbm,
                 acc, srcv, dstv, gbuf, sem):
    c = lax.axis_index("c")
    s = lax.axis_index("s")
    r0 = s * TPR
    pltpu.sync_copy(zeros_hbm, acc.at[pl.ds(r0, TPR)])
    pltpu.sync_copy(srcb_hbm.at[pl.ds(c * (EP // 128) + s * C1, C1)], srcv)
    pltpu.sync_copy(dst_hbm.at[pl.ds(s * C1, C1)], dstv)
    plsc.subcore_barrier()

    def body(i, carry):
        pltpu.async_copy(tab_hbm.at[srcv.at[i]], gbuf, sem).wait()
        pltpu.sync_copy(gbuf, acc.at[dstv.at[i]], add=True)
        return carry

    lax.fori_loop(0, C1, body, 0)
    plsc.subcore_barrier()
    pltpu.sync_copy(acc.at[pl.ds(r0, TPR)], out_hbm.at[pl.ds(c * NP + r0, TPR)])


@functools.partial(
    pl.kernel,
    out_type=jax.ShapeDtypeStruct((2 * NP, D2), jnp.float32),
    mesh=_mesh(),
    scratch_types=[
        pltpu.VMEM_SHARED((NP, D2), jnp.float32),
        pltpu.VMEM((C2, 128), jnp.int32),
        pltpu.VMEM((C2, 128), jnp.int32),
        pltpu.VMEM((128, D2), jnp.float32),
        pltpu.SemaphoreType.DMA,
    ],
)
def _agg2_kernel(tab_hbm, src_hbm, dst_hbm, zeros_hbm, out_hbm,
                 acc, srcv, dstv, gbuf, sem):
    c = lax.axis_index("c")
    s = lax.axis_index("s")
    r0 = s * TPR
    pltpu.sync_copy(zeros_hbm, acc.at[pl.ds(r0, TPR)])
    e0 = (c * 16 + s) * C2
    pltpu.sync_copy(src_hbm.at[pl.ds(e0, C2)], srcv)
    pltpu.sync_copy(dst_hbm.at[pl.ds(e0, C2)], dstv)
    plsc.subcore_barrier()

    def body(i, carry):
        pltpu.async_copy(tab_hbm.at[srcv.at[i]], gbuf, sem).wait()
        pltpu.sync_copy(gbuf, acc.at[dstv.at[i]], add=True)
        return carry

    lax.fori_loop(0, C2, body, 0)
    plsc.subcore_barrier()
    pltpu.sync_copy(acc.at[pl.ds(r0, TPR)], out_hbm.at[pl.ds(c * NP + r0, TPR)])


def _dinv(deg_ref):
    deg = deg_ref[0, :, 0:1] + deg_ref[1, :, 0:1] + 1.0
    return lax.rsqrt(deg)


def _k1_body(deg_ref, x_ref, w_ref, out_ref):
    dinv = _dinv(deg_ref)
    y = jnp.dot(x_ref[...], w_ref[...], preferred_element_type=jnp.float32)
    y = y * dinv
    out_ref[0] = y[:, :DH]
    out_ref[1] = y[:, DH:]


def _k1(degR, x, W1):
    return pl.pallas_call(
        _k1_body,
        grid=(NP // RB,),
        in_specs=[
            pl.BlockSpec((2, RB, 128), lambda i: (0, i, 0)),
            pl.BlockSpec((RB, D), lambda i: (i, 0)),
            pl.BlockSpec((D, D), lambda i: (0, 0)),
        ],
        out_specs=pl.BlockSpec((2, RB, DH), lambda i: (0, i, 0)),
        out_shape=jax.ShapeDtypeStruct((2, NP, DH), jnp.float32),
    )(degR, x, W1)


def _k2_body(deg_ref, agg_ref, xs_ref, b1_ref, w2_ref, x1_ref, xw2_ref):
    dinv = _dinv(deg_ref)
    h = jnp.concatenate(
        [agg_ref[0] + xs_ref[0], agg_ref[1] + xs_ref[1]], axis=1)
    x1 = jnp.maximum(dinv * h + b1_ref[...], 0.0)
    x1_ref[...] = x1
    y2 = jnp.dot(x1, w2_ref[...], preferred_element_type=jnp.float32)
    xw2_ref[...] = y2 * dinv


def _k2(degR, aggR, xsR, b1, W2p):
    return pl.pallas_call(
        _k2_body,
        grid=(NP // RB,),
        in_specs=[
            pl.BlockSpec((2, RB, 128), lambda i: (0, i, 0)),
            pl.BlockSpec((2, RB, DH), lambda i: (0, i, 0)),
            pl.BlockSpec((2, RB, DH), lambda i: (0, i, 0)),
            pl.BlockSpec((1, D), lambda i: (0, 0)),
            pl.BlockSpec((D, D2), lambda i: (0, 0)),
        ],
        out_specs=[
            pl.BlockSpec((RB, D), lambda i: (i, 0)),
            pl.BlockSpec((RB, D2), lambda i: (i, 0)),
        ],
        out_shape=[
            jax.ShapeDtypeStruct((NP, D), jnp.float32),
            jax.ShapeDtypeStruct((NP, D2), jnp.float32),
        ],
    )(degR, aggR, xsR, b1, W2p)


def _k3_body(deg_ref, agg_ref, xw2_ref, b2_ref, out_ref):
    dinv = _dinv(deg_ref)
    h = agg_ref[0] + agg_ref[1] + xw2_ref[...]
    out_ref[...] = dinv * h + b2_ref[...]


def _k3(degR, agg2R, xw2, b2p):
    return pl.pallas_call(
        _k3_body,
        grid=(NP // RB,),
        in_specs=[
            pl.BlockSpec((2, RB, 128), lambda i: (0, i, 0)),
            pl.BlockSpec((2, RB, D2), lambda i: (0, i, 0)),
            pl.BlockSpec((RB, D2), lambda i: (i, 0)),
            pl.BlockSpec((1, D2), lambda i: (0, 0)),
        ],
        out_specs=pl.BlockSpec((RB, D2), lambda i: (i, 0)),
        out_shape=jax.ShapeDtypeStruct((NP, D2), jnp.float32),
    )(degR, agg2R, xw2, b2p)


@jax.jit
def _run(x, edge_index, W1, b1, W2, b2):
    ei = edge_index.astype(jnp.int32)
    pad_idx = N + (jnp.arange(EP - E, dtype=jnp.int32) % (NP - N))
    src = jnp.concatenate([ei[0], pad_idx])
    dst = jnp.concatenate([ei[1], pad_idx])
    src2d = src.reshape(EP // 128, 128)
    dst2d = dst.reshape(EP // 128, 128)
    srcb = jnp.concatenate([src2d, src2d + NP], axis=0)
    xp = jnp.pad(x, ((0, NP - N), (0, 0)))
    W2p = jnp.pad(W2, ((0, 0), (0, D2 - W2.shape[1])))
    b2p = jnp.pad(b2, (0, D2 - b2.shape[0])).reshape(1, D2)
    ones = jnp.ones((128, 128), jnp.float32)
    zeros128 = jnp.zeros((TPR, DH), jnp.float32)

    degf = _deg_kernel(dst2d, ones, zeros128)
    degR = degf.reshape(2, NP, 128)
    xs1 = _k1(degR, xp, W1)
    agg1 = _agg1_kernel(xs1.reshape(2 * NP, DH), srcb, dst2d, zeros128)
    x1, xw2 = _k2(degR, agg1.reshape(2, NP, DH), xs1,
                  b1.reshape(1, D), W2p)
    agg2 = _agg2_kernel(xw2, src2d, dst2d, zeros128)
    x2 = _k3(degR, agg2.reshape(2, NP, D2), xw2, b2p)
    return x1[:N], x2[:N, :W2.shape[1]]


def kernel(x, edge_index, W1, b1, W2, b2):
    return _run(x, edge_index, W1, b1, W2, b2)


# speedup vs baseline: 15.9944x; 1.2330x over previous
import functools

import jax
import jax.numpy as jnp
from jax import lax
from jax.experimental import pallas as pl
from jax.experimental.pallas import tpu as pltpu
from jax.experimental.pallas import tpu_sc as plsc

N = 10000
NP = 10240
E = 160000
EP = 163840
D = 256
DH = 128
D2 = 128
RB = 2048
TPR = NP // 16
C1 = 80
C2 = 40
NCH = EP // 128


def _mesh():
    return plsc.VectorSubcoreMesh(core_axis_name="c", subcore_axis_name="s")


@functools.partial(
    pl.kernel,
    out_type=jax.ShapeDtypeStruct((2 * NP, 128), jnp.float32),
    mesh=_mesh(),
    scratch_types=[
        pltpu.VMEM_SHARED((NP, 128), jnp.float32),
        pltpu.VMEM((C2, 128), jnp.int32),
        pltpu.VMEM((128, 128), jnp.float32),
        pltpu.SemaphoreType.DMA,
    ],
)
def _deg_kernel(dst_hbm, ones_hbm, zeros_hbm, out_hbm, acc, dstv, onesv, sem):
    c = lax.axis_index("c")
    s = lax.axis_index("s")
    r0 = s * TPR
    pltpu.sync_copy(zeros_hbm, acc.at[pl.ds(r0, TPR)])
    pltpu.sync_copy(ones_hbm, onesv)
    pltpu.sync_copy(dst_hbm.at[pl.ds((c * 16 + s) * C2, C2)], dstv)
    plsc.subcore_barrier()

    def body(g, carry):
        for b in range(8):
            pltpu.async_copy(onesv, acc.at[dstv.at[g * 8 + b]], sem, add=True)
        for b in range(8):
            pltpu.make_async_copy(onesv, acc.at[dstv.at[0]], sem).wait()
        return carry

    lax.fori_loop(0, C2 // 8, body, 0)
    plsc.subcore_barrier()
    pltpu.sync_copy(acc.at[pl.ds(r0, TPR)], out_hbm.at[pl.ds(c * NP + r0, TPR)])


def _agg_ring(tab_hbm, sd_hbm, base, acc, idxv, gbuf, gsems, ssems, isems,
              nchunks):
    for j in range(4):
        pltpu.async_copy(sd_hbm.at[base + j], idxv.at[j], isems[j])
    for b in range(2):
        pltpu.make_async_copy(sd_hbm.at[base], idxv.at[b], isems[b]).wait()
        pltpu.async_copy(tab_hbm.at[idxv.at[b, 0]], gbuf.at[b], gsems[b])

    def group(g, carry):
        for b in range(4):
            c = g * 4 + b
            bb = b % 2
            pb = 1 - bb
            jn = (b + 1) % 4
            jr = (b + 3) % 4
            pltpu.make_async_copy(tab_hbm.at[idxv.at[0, 0]], gbuf.at[bb],
                                  gsems[bb]).wait()
            pltpu.async_copy(gbuf.at[bb], acc.at[idxv.at[b, 1]], ssems[bb],
                             add=True)

            @pl.when(c >= 1)
            def _():
                pltpu.make_async_copy(gbuf.at[pb], acc.at[idxv.at[0, 1]],
                                      ssems[pb]).wait()

            @pl.when((c >= 1) & (c + 3 < nchunks))
            def _():
                pltpu.async_copy(sd_hbm.at[base + c + 3], idxv.at[jr],
                                 isems[jr])

            @pl.when((c >= 1) & (c + 1 < nchunks))
            def _():
                pltpu.make_async_copy(sd_hbm.at[base], idxv.at[jn],
                                      isems[jn]).wait()
                pltpu.async_copy(tab_hbm.at[idxv.at[jn, 0]], gbuf.at[pb],
                                 gsems[pb])
        return carry

    lax.fori_loop(0, nchunks // 4, group, 0)
    pltpu.make_async_copy(gbuf.at[(nchunks - 1) % 2], acc.at[idxv.at[0, 1]],
                          ssems[(nchunks - 1) % 2]).wait()


_AGG_SCRATCH = [
    pltpu.VMEM((4, 2, 128), jnp.int32),
    pltpu.VMEM((2, 128, 128), jnp.float32),
    pltpu.SemaphoreType.DMA,
    pltpu.SemaphoreType.DMA,
    pltpu.SemaphoreType.DMA,
    pltpu.SemaphoreType.DMA,
    pltpu.SemaphoreType.DMA,
    pltpu.SemaphoreType.DMA,
    pltpu.SemaphoreType.DMA,
    pltpu.SemaphoreType.DMA,
]


@functools.partial(
    pl.kernel,
    out_type=jax.ShapeDtypeStruct((2 * NP, DH), jnp.float32),
    mesh=_mesh(),
    scratch_types=[pltpu.VMEM_SHARED((NP, DH), jnp.float32)] + _AGG_SCRATCH,
)
def _agg1_kernel(tab_hbm, sd_hbm, zeros_hbm, out_hbm,
                 acc, idxv, gbuf, g0, g1, s0, s1, i0, i1, i2, i3):
    c = lax.axis_index("c")
    s = lax.axis_index("s")
    r0 = s * TPR
    pltpu.sync_copy(zeros_hbm, acc.at[pl.ds(r0, TPR)])
    plsc.subcore_barrier()
    _agg_ring(tab_hbm, sd_hbm, c * NCH + s * C1, acc, idxv, gbuf,
              [g0, g1], [s0, s1], [i0, i1, i2, i3], C1)
    plsc.subcore_barrier()
    pltpu.sync_copy(acc.at[pl.ds(r0, TPR)], out_hbm.at[pl.ds(c * NP + r0, TPR)])


@functools.partial(
    pl.kernel,
    out_type=jax.ShapeDtypeStruct((2 * NP, D2), jnp.float32),
    mesh=_mesh(),
    scratch_types=[pltpu.VMEM_SHARED((NP, D2), jnp.float32)] + _AGG_SCRATCH,
)
def _agg2_kernel(tab_hbm, sd_hbm, zeros_hbm, out_hbm,
                 acc, idxv, gbuf, g0, g1, s0, s1, i0, i1, i2, i3):
    c = lax.axis_index("c")
    s = lax.axis_index("s")
    r0 = s * TPR
    pltpu.sync_copy(zeros_hbm, acc.at[pl.ds(r0, TPR)])
    plsc.subcore_barrier()
    _agg_ring(tab_hbm, sd_hbm, (c * 16 + s) * C2, acc, idxv, gbuf,
              [g0, g1], [s0, s1], [i0, i1, i2, i3], C2)
    plsc.subcore_barrier()
    pltpu.sync_copy(acc.at[pl.ds(r0, TPR)], out_hbm.at[pl.ds(c * NP + r0, TPR)])


def _dinv(deg_ref):
    deg = deg_ref[0, :, 0:1] + deg_ref[1, :, 0:1] + 1.0
    return lax.rsqrt(deg)


def _k1_body(deg_ref, x_ref, w_ref, out_ref):
    dinv = _dinv(deg_ref)
    y = jnp.dot(x_ref[...], w_ref[...], preferred_element_type=jnp.float32)
    y = y * dinv
    out_ref[0] = y[:, :DH]
    out_ref[1] = y[:, DH:]


def _k1(degR, x, W1):
    return pl.pallas_call(
        _k1_body,
        grid=(NP // RB,),
        in_specs=[
            pl.BlockSpec((2, RB, 128), lambda i: (0, i, 0)),
            pl.BlockSpec((RB, D), lambda i: (i, 0)),
            pl.BlockSpec((D, D), lambda i: (0, 0)),
        ],
        out_specs=pl.BlockSpec((2, RB, DH), lambda i: (0, i, 0)),
        out_shape=jax.ShapeDtypeStruct((2, NP, DH), jnp.float32),
    )(degR, x, W1)


def _k2_body(deg_ref, agg_ref, xs_ref, b1_ref, w2_ref, x1_ref, xw2_ref):
    dinv = _dinv(deg_ref)
    h = jnp.concatenate(
        [agg_ref[0] + xs_ref[0], agg_ref[1] + xs_ref[1]], axis=1)
    x1 = jnp.maximum(dinv * h + b1_ref[...], 0.0)
    x1_ref[...] = x1
    y2 = jnp.dot(x1, w2_ref[...], preferred_element_type=jnp.float32)
    xw2_ref[...] = y2 * dinv


def _k2(degR, aggR, xsR, b1, W2p):
    return pl.pallas_call(
        _k2_body,
        grid=(NP // RB,),
        in_specs=[
            pl.BlockSpec((2, RB, 128), lambda i: (0, i, 0)),
            pl.BlockSpec((2, RB, DH), lambda i: (0, i, 0)),
            pl.BlockSpec((2, RB, DH), lambda i: (0, i, 0)),
            pl.BlockSpec((1, D), lambda i: (0, 0)),
            pl.BlockSpec((D, D2), lambda i: (0, 0)),
        ],
        out_specs=[
            pl.BlockSpec((RB, D), lambda i: (i, 0)),
            pl.BlockSpec((RB, D2), lambda i: (i, 0)),
        ],
        out_shape=[
            jax.ShapeDtypeStruct((NP, D), jnp.float32),
            jax.ShapeDtypeStruct((NP, D2), jnp.float32),
        ],
    )(degR, aggR, xsR, b1, W2p)


def _k3_body(deg_ref, agg_ref, xw2_ref, b2_ref, out_ref):
    dinv = _dinv(deg_ref)
    h = agg_ref[0] + agg_ref[1] + xw2_ref[...]
    out_ref[...] = dinv * h + b2_ref[...]


def _k3(degR, agg2R, xw2, b2p):
    return pl.pallas_call(
        _k3_body,
        grid=(NP // RB,),
        in_specs=[
            pl.BlockSpec((2, RB, 128), lambda i: (0, i, 0)),
            pl.BlockSpec((2, RB, D2), lambda i: (0, i, 0)),
            pl.BlockSpec((RB, D2), lambda i: (i, 0)),
            pl.BlockSpec((1, D2), lambda i: (0, 0)),
        ],
        out_specs=pl.BlockSpec((RB, D2), lambda i: (i, 0)),
        out_shape=jax.ShapeDtypeStruct((NP, D2), jnp.float32),
    )(degR, agg2R, xw2, b2p)


@jax.jit
def _run(x, edge_index, W1, b1, W2, b2):
    ei = edge_index.astype(jnp.int32)
    pad_idx = N + (jnp.arange(EP - E, dtype=jnp.int32) % (NP - N))
    src = jnp.concatenate([ei[0], pad_idx])
    dst = jnp.concatenate([ei[1], pad_idx])
    src2d = src.reshape(NCH, 128)
    dst2d = dst.reshape(NCH, 128)
    sd = jnp.stack([src2d, dst2d], axis=1)
    sd1 = jnp.concatenate(
        [sd, jnp.stack([src2d + NP, dst2d], axis=1)], axis=0)
    xp = jnp.pad(x, ((0, NP - N), (0, 0)))
    W2p = jnp.pad(W2, ((0, 0), (0, D2 - W2.shape[1])))
    b2p = jnp.pad(b2, (0, D2 - b2.shape[0])).reshape(1, D2)
    ones = jnp.ones((128, 128), jnp.float32)
    zeros128 = jnp.zeros((TPR, DH), jnp.float32)

    degf = _deg_kernel(dst2d, ones, zeros128)
    degR = degf.reshape(2, NP, 128)
    xs1 = _k1(degR, xp, W1)
    agg1 = _agg1_kernel(xs1.reshape(2 * NP, DH), sd1, zeros128)
    x1, xw2 = _k2(degR, agg1.reshape(2, NP, DH), xs1,
                  b1.reshape(1, D), W2p)
    agg2 = _agg2_kernel(xw2, sd, zeros128)
    x2 = _k3(degR, agg2.reshape(2, NP, D2), xw2, b2p)
    return x1[:N], x2[:N, :W2.shape[1]]


def kernel(x, edge_index, W1, b1, W2, b2):
    return _run(x, edge_index, W1, b1, W2, b2)


# speedup vs baseline: 16.7773x; 1.0490x over previous
import functools

import jax
import jax.numpy as jnp
from jax import lax
from jax.experimental import pallas as pl
from jax.experimental.pallas import tpu as pltpu
from jax.experimental.pallas import tpu_sc as plsc

N = 10000
NP = 10240
E = 160000
EP = 163840
D = 256
DH = 128
D2 = 128
RB = 2048
TPR = NP // 16
C1 = 80
C2 = 40
NCH = EP // 128


def _mesh():
    return plsc.VectorSubcoreMesh(core_axis_name="c", subcore_axis_name="s")


EB = 16384


def _deg_body(dst_ref, out_ref):
    c = jnp.zeros((80, 128), jnp.float32)
    for r in range(EP // EB):
        d = dst_ref[r]
        hi = jax.lax.shift_right_logical(d, 7)
        lo = jax.lax.bitwise_and(d, 127)
        a = (jax.lax.broadcasted_iota(jnp.int32, (80, EB), 0) == hi[None, :]
             ).astype(jnp.bfloat16)
        b = (jax.lax.broadcasted_iota(jnp.int32, (128, EB), 0) == lo[None, :]
             ).astype(jnp.bfloat16)
        c = c + jax.lax.dot_general(a, b, (((1,), (1,)), ((), ())),
                                    preferred_element_type=jnp.float32)
    out_ref[...] = c


def _deg_tc(dstE):
    return pl.pallas_call(
        _deg_body,
        out_shape=jax.ShapeDtypeStruct((80, 128), jnp.float32),
    )(dstE)


def _agg_ring(tab_hbm, sd_hbm, base, acc, idxv, gbuf, gsems, ssems, isems,
              nchunks):
    for j in range(4):
        pltpu.async_copy(sd_hbm.at[base + j], idxv.at[j], isems[j])
    for b in range(2):
        pltpu.make_async_copy(sd_hbm.at[base], idxv.at[b], isems[b]).wait()
        pltpu.async_copy(tab_hbm.at[idxv.at[b, 0]], gbuf.at[b], gsems[b])

    def group(g, carry):
        for b in range(4):
            c = g * 4 + b
            bb = b % 2
            pb = 1 - bb
            jn = (b + 1) % 4
            jr = (b + 3) % 4
            pltpu.make_async_copy(tab_hbm.at[idxv.at[0, 0]], gbuf.at[bb],
                                  gsems[bb]).wait()
            pltpu.async_copy(gbuf.at[bb], acc.at[idxv.at[b, 1]], ssems[bb],
                             add=True)

            @pl.when(c >= 1)
            def _():
                pltpu.make_async_copy(gbuf.at[pb], acc.at[idxv.at[0, 1]],
                                      ssems[pb]).wait()

            @pl.when((c >= 1) & (c + 3 < nchunks))
            def _():
                pltpu.async_copy(sd_hbm.at[base + c + 3], idxv.at[jr],
                                 isems[jr])

            @pl.when((c >= 1) & (c + 1 < nchunks))
            def _():
                pltpu.make_async_copy(sd_hbm.at[base], idxv.at[jn],
                                      isems[jn]).wait()
                pltpu.async_copy(tab_hbm.at[idxv.at[jn, 0]], gbuf.at[pb],
                                 gsems[pb])
        return carry

    lax.fori_loop(0, nchunks // 4, group, 0)
    pltpu.make_async_copy(gbuf.at[(nchunks - 1) % 2], acc.at[idxv.at[0, 1]],
                          ssems[(nchunks - 1) % 2]).wait()


_AGG_SCRATCH = [
    pltpu.VMEM((4, 2, 128), jnp.int32),
    pltpu.VMEM((2, 128, 128), jnp.float32),
    pltpu.SemaphoreType.DMA,
    pltpu.SemaphoreType.DMA,
    pltpu.SemaphoreType.DMA,
    pltpu.SemaphoreType.DMA,
    pltpu.SemaphoreType.DMA,
    pltpu.SemaphoreType.DMA,
    pltpu.SemaphoreType.DMA,
    pltpu.SemaphoreType.DMA,
]


@functools.partial(
    pl.kernel,
    out_type=jax.ShapeDtypeStruct((2 * NP, DH), jnp.float32),
    mesh=_mesh(),
    scratch_types=[pltpu.VMEM_SHARED((NP, DH), jnp.float32)] + _AGG_SCRATCH,
)
def _agg1_kernel(tab_hbm, sd_hbm, zeros_hbm, out_hbm,
                 acc, idxv, gbuf, g0, g1, s0, s1, i0, i1, i2, i3):
    c = lax.axis_index("c")
    s = lax.axis_index("s")
    r0 = s * TPR
    pltpu.sync_copy(zeros_hbm, acc.at[pl.ds(r0, TPR)])
    plsc.subcore_barrier()
    _agg_ring(tab_hbm, sd_hbm, c * NCH + s * C1, acc, idxv, gbuf,
              [g0, g1], [s0, s1], [i0, i1, i2, i3], C1)
    plsc.subcore_barrier()
    pltpu.sync_copy(acc.at[pl.ds(r0, TPR)], out_hbm.at[pl.ds(c * NP + r0, TPR)])


@functools.partial(
    pl.kernel,
    out_type=jax.ShapeDtypeStruct((2 * NP, D2), jnp.float32),
    mesh=_mesh(),
    scratch_types=[pltpu.VMEM_SHARED((NP, D2), jnp.float32)] + _AGG_SCRATCH,
)
def _agg2_kernel(tab_hbm, sd_hbm, zeros_hbm, out_hbm,
                 acc, idxv, gbuf, g0, g1, s0, s1, i0, i1, i2, i3):
    c = lax.axis_index("c")
    s = lax.axis_index("s")
    r0 = s * TPR
    pltpu.sync_copy(zeros_hbm, acc.at[pl.ds(r0, TPR)])
    plsc.subcore_barrier()
    _agg_ring(tab_hbm, sd_hbm, (c * 16 + s) * C2, acc, idxv, gbuf,
              [g0, g1], [s0, s1], [i0, i1, i2, i3], C2)
    plsc.subcore_barrier()
    pltpu.sync_copy(acc.at[pl.ds(r0, TPR)], out_hbm.at[pl.ds(c * NP + r0, TPR)])


def _dinv(deg_ref):
    return lax.rsqrt(deg_ref[...] + 1.0)


def _k1_body(deg_ref, x_ref, w_ref, out_ref):
    dinv = _dinv(deg_ref)
    y = jnp.dot(x_ref[...], w_ref[...], preferred_element_type=jnp.float32)
    y = y * dinv
    out_ref[0] = y[:, :DH]
    out_ref[1] = y[:, DH:]


def _k1(degR, x, W1):
    return pl.pallas_call(
        _k1_body,
        grid=(NP // RB,),
        in_specs=[
            pl.BlockSpec((RB, 1), lambda i: (i, 0)),
            pl.BlockSpec((RB, D), lambda i: (i, 0)),
            pl.BlockSpec((D, D), lambda i: (0, 0)),
        ],
        out_specs=pl.BlockSpec((2, RB, DH), lambda i: (0, i, 0)),
        out_shape=jax.ShapeDtypeStruct((2, NP, DH), jnp.float32),
    )(degR, x, W1)


def _k2_body(deg_ref, agg_ref, xs_ref, b1_ref, w2_ref, x1_ref, xw2_ref):
    dinv = _dinv(deg_ref)
    h = jnp.concatenate(
        [agg_ref[0] + xs_ref[0], agg_ref[1] + xs_ref[1]], axis=1)
    x1 = jnp.maximum(dinv * h + b1_ref[...], 0.0)
    x1_ref[...] = x1
    y2 = jnp.dot(x1, w2_ref[...], preferred_element_type=jnp.float32)
    xw2_ref[...] = y2 * dinv


def _k2(degR, aggR, xsR, b1, W2p):
    return pl.pallas_call(
        _k2_body,
        grid=(NP // RB,),
        in_specs=[
            pl.BlockSpec((RB, 1), lambda i: (i, 0)),
            pl.BlockSpec((2, RB, DH), lambda i: (0, i, 0)),
            pl.BlockSpec((2, RB, DH), lambda i: (0, i, 0)),
            pl.BlockSpec((1, D), lambda i: (0, 0)),
            pl.BlockSpec((D, D2), lambda i: (0, 0)),
        ],
        out_specs=[
            pl.BlockSpec((RB, D), lambda i: (i, 0)),
            pl.BlockSpec((RB, D2), lambda i: (i, 0)),
        ],
        out_shape=[
            jax.ShapeDtypeStruct((NP, D), jnp.float32),
            jax.ShapeDtypeStruct((NP, D2), jnp.float32),
        ],
    )(degR, aggR, xsR, b1, W2p)


def _k3_body(deg_ref, agg_ref, xw2_ref, b2_ref, out_ref):
    dinv = _dinv(deg_ref)
    h = agg_ref[0] + agg_ref[1] + xw2_ref[...]
    out_ref[...] = dinv * h + b2_ref[...]


def _k3(degR, agg2R, xw2, b2p):
    return pl.pallas_call(
        _k3_body,
        grid=(NP // RB,),
        in_specs=[
            pl.BlockSpec((RB, 1), lambda i: (i, 0)),
            pl.BlockSpec((2, RB, D2), lambda i: (0, i, 0)),
            pl.BlockSpec((RB, D2), lambda i: (i, 0)),
            pl.BlockSpec((1, D2), lambda i: (0, 0)),
        ],
        out_specs=pl.BlockSpec((RB, D2), lambda i: (i, 0)),
        out_shape=jax.ShapeDtypeStruct((NP, D2), jnp.float32),
    )(degR, agg2R, xw2, b2p)


@jax.jit
def _run(x, edge_index, W1, b1, W2, b2):
    ei = edge_index.astype(jnp.int32)
    pad_idx = N + (jnp.arange(EP - E, dtype=jnp.int32) % (NP - N))
    src = jnp.concatenate([ei[0], pad_idx])
    dst = jnp.concatenate([ei[1], pad_idx])
    src2d = src.reshape(NCH, 128)
    dst2d = dst.reshape(NCH, 128)
    sd = jnp.stack([src2d, dst2d], axis=1)
    sd1 = jnp.concatenate(
        [sd, jnp.stack([src2d + NP, dst2d], axis=1)], axis=0)
    xp = jnp.pad(x, ((0, NP - N), (0, 0)))
    W2p = jnp.pad(W2, ((0, 0), (0, D2 - W2.shape[1])))
    b2p = jnp.pad(b2, (0, D2 - b2.shape[0])).reshape(1, D2)
    zeros128 = jnp.zeros((TPR, DH), jnp.float32)

    degC = _deg_tc(dst.reshape(EP // EB, EB))
    degR = degC.reshape(NP, 1)
    xs1 = _k1(degR, xp, W1)
    agg1 = _agg1_kernel(xs1.reshape(2 * NP, DH), sd1, zeros128)
    x1, xw2 = _k2(degR, agg1.reshape(2, NP, DH), xs1,
                  b1.reshape(1, D), W2p)
    agg2 = _agg2_kernel(xw2, sd, zeros128)
    x2 = _k3(degR, agg2.reshape(2, NP, D2), xw2, b2p)
    return x1[:N], x2[:N, :W2.shape[1]]


def kernel(x, edge_index, W1, b1, W2, b2):
    return _run(x, edge_index, W1, b1, W2, b2)


# speedup vs baseline: 17.4933x; 1.0427x over previous
import functools

import jax
import jax.numpy as jnp
from jax import lax
from jax.experimental import pallas as pl
from jax.experimental.pallas import tpu as pltpu
from jax.experimental.pallas import tpu_sc as plsc

N = 10000
NP = 10240
E = 160000
EP = 163840
D = 256
DH = 128
D2 = 128
RB = 2048
TPR = NP // 16
C1 = 80
C2 = 40
NCH = EP // 128


def _mesh():
    return plsc.VectorSubcoreMesh(core_axis_name="c", subcore_axis_name="s")


EB = 16384


def _count_hi_lo(dst_ref):
    c = jnp.zeros((80, 128), jnp.float32)
    for r in range(EP // EB):
        d = dst_ref[r]
        hi = jax.lax.shift_right_logical(d, 7)
        lo = jax.lax.bitwise_and(d, 127)
        a = (jax.lax.broadcasted_iota(jnp.int32, (80, EB), 0) == hi[None, :]
             ).astype(jnp.bfloat16)
        b = (jax.lax.broadcasted_iota(jnp.int32, (128, EB), 0) == lo[None, :]
             ).astype(jnp.bfloat16)
        c = c + jax.lax.dot_general(a, b, (((1,), (1,)), ((), ())),
                                    preferred_element_type=jnp.float32)
    return c


def _dinv3(c_blk):
    return lax.rsqrt(c_blk + 1.0)[:, :, None]


def _agg_ring(tab_hbm, sd_hbm, zeros_hbm, out_hbm, r0, cid, base, acc, idxv,
              gbuf, gsems, ssems, isems, zsem, nchunks):
    pltpu.async_copy(zeros_hbm, acc.at[pl.ds(r0, TPR)], zsem)
    for j in range(4):
        pltpu.async_copy(sd_hbm.at[base + j], idxv.at[j], isems[j])
    for b in range(2):
        pltpu.make_async_copy(sd_hbm.at[base], idxv.at[b], isems[b]).wait()
        pltpu.async_copy(tab_hbm.at[idxv.at[b, 0]], gbuf.at[b], gsems[b])
    pltpu.make_async_copy(zeros_hbm, acc.at[pl.ds(r0, TPR)], zsem).wait()
    plsc.subcore_barrier()

    def group(g, carry):
        for b in range(4):
            c = g * 4 + b
            bb = b % 2
            pb = 1 - bb
            jn = (b + 1) % 4
            jr = (b + 3) % 4
            pltpu.make_async_copy(tab_hbm.at[idxv.at[0, 0]], gbuf.at[bb],
                                  gsems[bb]).wait()
            pltpu.async_copy(gbuf.at[bb], acc.at[idxv.at[b, 1]], ssems[bb],
                             add=True)

            @pl.when(c >= 1)
            def _():
                pltpu.make_async_copy(gbuf.at[pb], acc.at[idxv.at[0, 1]],
                                      ssems[pb]).wait()

            @pl.when((c >= 1) & (c + 3 < nchunks))
            def _():
                pltpu.async_copy(sd_hbm.at[base + c + 3], idxv.at[jr],
                                 isems[jr])

            @pl.when((c >= 1) & (c + 1 < nchunks))
            def _():
                pltpu.make_async_copy(sd_hbm.at[base], idxv.at[jn],
                                      isems[jn]).wait()
                pltpu.async_copy(tab_hbm.at[idxv.at[jn, 0]], gbuf.at[pb],
                                 gsems[pb])
        return carry

    lax.fori_loop(0, nchunks // 4, group, 0)
    pltpu.make_async_copy(gbuf.at[(nchunks - 1) % 2], acc.at[idxv.at[0, 1]],
                          ssems[(nchunks - 1) % 2]).wait()
    plsc.subcore_barrier()
    pltpu.sync_copy(acc.at[pl.ds(r0, TPR)],
                    out_hbm.at[pl.ds(cid * NP + r0, TPR)])


_AGG_SCRATCH = [
    pltpu.VMEM((4, 2, 128), jnp.int32),
    pltpu.VMEM((2, 128, 128), jnp.float32),
    pltpu.SemaphoreType.DMA,
    pltpu.SemaphoreType.DMA,
    pltpu.SemaphoreType.DMA,
    pltpu.SemaphoreType.DMA,
    pltpu.SemaphoreType.DMA,
    pltpu.SemaphoreType.DMA,
    pltpu.SemaphoreType.DMA,
    pltpu.SemaphoreType.DMA,
    pltpu.SemaphoreType.DMA,
]


@functools.partial(
    pl.kernel,
    out_type=jax.ShapeDtypeStruct((2 * NP, DH), jnp.float32),
    mesh=_mesh(),
    scratch_types=[pltpu.VMEM_SHARED((NP, DH), jnp.float32)] + _AGG_SCRATCH,
)
def _agg1_kernel(tab_hbm, sd_hbm, zeros_hbm, out_hbm,
                 acc, idxv, gbuf, g0, g1, s0, s1, i0, i1, i2, i3, z0):
    c = lax.axis_index("c")
    s = lax.axis_index("s")
    _agg_ring(tab_hbm, sd_hbm, zeros_hbm, out_hbm, s * TPR, c,
              c * NCH + s * C1, acc, idxv, gbuf,
              [g0, g1], [s0, s1], [i0, i1, i2, i3], z0, C1)


@functools.partial(
    pl.kernel,
    out_type=jax.ShapeDtypeStruct((2 * NP, D2), jnp.float32),
    mesh=_mesh(),
    scratch_types=[pltpu.VMEM_SHARED((NP, D2), jnp.float32)] + _AGG_SCRATCH,
)
def _agg2_kernel(tab_hbm, sd_hbm, zeros_hbm, out_hbm,
                 acc, idxv, gbuf, g0, g1, s0, s1, i0, i1, i2, i3, z0):
    c = lax.axis_index("c")
    s = lax.axis_index("s")
    _agg_ring(tab_hbm, sd_hbm, zeros_hbm, out_hbm, s * TPR, c,
              (c * 16 + s) * C2, acc, idxv, gbuf,
              [g0, g1], [s0, s1], [i0, i1, i2, i3], z0, C2)


def _k1_body(dst_ref, x_ref, w_ref, cout_ref, out_ref, cscr):
    i = pl.program_id(0)

    @pl.when(i == 0)
    def _():
        c = _count_hi_lo(dst_ref)
        cscr[...] = c
        cout_ref[...] = c

    dinv3 = _dinv3(cscr[pl.ds(i * 16, 16)])
    y = jnp.dot(x_ref[...], w_ref[...], preferred_element_type=jnp.float32)
    y = (y.reshape(16, 128, D) * dinv3).reshape(RB, D)
    out_ref[0] = y[:, :DH]
    out_ref[1] = y[:, DH:]


def _k1(dstE, x, W1):
    return pl.pallas_call(
        _k1_body,
        grid=(NP // RB,),
        in_specs=[
            pl.BlockSpec((EP // EB, EB), lambda i: (0, 0)),
            pl.BlockSpec((RB, D), lambda i: (i, 0)),
            pl.BlockSpec((D, D), lambda i: (0, 0)),
        ],
        out_specs=[
            pl.BlockSpec((80, 128), lambda i: (0, 0)),
            pl.BlockSpec((2, RB, DH), lambda i: (0, i, 0)),
        ],
        out_shape=[
            jax.ShapeDtypeStruct((80, 128), jnp.float32),
            jax.ShapeDtypeStruct((2, NP, DH), jnp.float32),
        ],
        scratch_shapes=[pltpu.VMEM((80, 128), jnp.float32)],
    )(dstE, x, W1)


def _k2_body(c_ref, agg_ref, xs_ref, b1_ref, w2_ref, x1_ref, xw2_ref):
    i = pl.program_id(0)
    dinv3 = _dinv3(c_ref[pl.ds(i * 16, 16)])
    h = jnp.concatenate(
        [agg_ref[0] + xs_ref[0], agg_ref[1] + xs_ref[1]], axis=1)
    h = (h.reshape(16, 128, D) * dinv3).reshape(RB, D)
    x1 = jnp.maximum(h + b1_ref[...], 0.0)
    x1_ref[...] = x1
    y2 = jnp.dot(x1, w2_ref[...], preferred_element_type=jnp.float32)
    xw2_ref[...] = (y2.reshape(16, 128, D2) * dinv3).reshape(RB, D2)


def _k2(degC, aggR, xsR, b1, W2p):
    return pl.pallas_call(
        _k2_body,
        grid=(NP // RB,),
        in_specs=[
            pl.BlockSpec((80, 128), lambda i: (0, 0)),
            pl.BlockSpec((2, RB, DH), lambda i: (0, i, 0)),
            pl.BlockSpec((2, RB, DH), lambda i: (0, i, 0)),
            pl.BlockSpec((1, D), lambda i: (0, 0)),
            pl.BlockSpec((D, D2), lambda i: (0, 0)),
        ],
        out_specs=[
            pl.BlockSpec((RB, D), lambda i: (i, 0)),
            pl.BlockSpec((RB, D2), lambda i: (i, 0)),
        ],
        out_shape=[
            jax.ShapeDtypeStruct((NP, D), jnp.float32),
            jax.ShapeDtypeStruct((NP, D2), jnp.float32),
        ],
    )(degC, aggR, xsR, b1, W2p)


def _k3_body(c_ref, agg_ref, xw2_ref, b2_ref, out_ref):
    i = pl.program_id(0)
    dinv3 = _dinv3(c_ref[pl.ds(i * 16, 16)])
    h = agg_ref[0] + agg_ref[1] + xw2_ref[...]
    h = (h.reshape(16, 128, D2) * dinv3).reshape(RB, D2)
    out_ref[...] = h + b2_ref[...]


def _k3(degC, agg2R, xw2, b2p):
    return pl.pallas_call(
        _k3_body,
        grid=(NP // RB,),
        in_specs=[
            pl.BlockSpec((80, 128), lambda i: (0, 0)),
            pl.BlockSpec((2, RB, D2), lambda i: (0, i, 0)),
            pl.BlockSpec((RB, D2), lambda i: (i, 0)),
            pl.BlockSpec((1, D2), lambda i: (0, 0)),
        ],
        out_specs=pl.BlockSpec((RB, D2), lambda i: (i, 0)),
        out_shape=jax.ShapeDtypeStruct((NP, D2), jnp.float32),
    )(degC, agg2R, xw2, b2p)


@jax.jit
def _run(x, edge_index, W1, b1, W2, b2):
    ei = edge_index.astype(jnp.int32)
    pad_idx = N + (jnp.arange(EP - E, dtype=jnp.int32) % (NP - N))
    src = jnp.concatenate([ei[0], pad_idx])
    dst = jnp.concatenate([ei[1], pad_idx])
    src2d = src.reshape(NCH, 128)
    dst2d = dst.reshape(NCH, 128)
    sd = jnp.stack([src2d, dst2d], axis=1)
    sd1 = jnp.concatenate(
        [sd, jnp.stack([src2d + NP, dst2d], axis=1)], axis=0)
    xp = jnp.pad(x, ((0, NP - N), (0, 0)))
    W2p = jnp.pad(W2, ((0, 0), (0, D2 - W2.shape[1])))
    b2p = jnp.pad(b2, (0, D2 - b2.shape[0])).reshape(1, D2)
    zeros128 = jnp.zeros((TPR, DH), jnp.float32)

    degC, xs1 = _k1(dst.reshape(EP // EB, EB), xp, W1)
    agg1 = _agg1_kernel(xs1.reshape(2 * NP, DH), sd1, zeros128)
    x1, xw2 = _k2(degC, agg1.reshape(2, NP, DH), xs1,
                  b1.reshape(1, D), W2p)
    agg2 = _agg2_kernel(xw2, sd, zeros128)
    x2 = _k3(degC, agg2.reshape(2, NP, D2), xw2, b2p)
    return x1[:N], x2[:N, :W2.shape[1]]


def kernel(x, edge_index, W1, b1, W2, b2):
    return _run(x, edge_index, W1, b1, W2, b2)


# speedup vs baseline: 17.7829x; 1.0166x over previous
import functools

import jax
import jax.numpy as jnp
from jax import lax
from jax.experimental import pallas as pl
from jax.experimental.pallas import tpu as pltpu
from jax.experimental.pallas import tpu_sc as plsc

N = 10000
NP = 10240
E = 160000
EP = 163840
D = 256
DH = 128
D2 = 128
RB = 2048
TPR = NP // 16
C1 = 80
C2 = 40
NCH = EP // 128


def _mesh():
    return plsc.VectorSubcoreMesh(core_axis_name="c", subcore_axis_name="s")


EB = 16384


def _count_hi_lo(dst_ref):
    c = jnp.zeros((80, 128), jnp.float32)
    for r in range(EP // EB):
        d = dst_ref[r]
        hi = jax.lax.shift_right_logical(d, 7)
        lo = jax.lax.bitwise_and(d, 127)
        a = (jax.lax.broadcasted_iota(jnp.int32, (80, EB), 0) == hi[None, :]
             ).astype(jnp.bfloat16)
        b = (jax.lax.broadcasted_iota(jnp.int32, (128, EB), 0) == lo[None, :]
             ).astype(jnp.bfloat16)
        c = c + jax.lax.dot_general(a, b, (((1,), (1,)), ((), ())),
                                    preferred_element_type=jnp.float32)
    return c


def _dinv3(c_blk):
    return lax.rsqrt(c_blk + 1.0)[:, :, None]


def _agg_ring(tab_hbm, sd_hbm, zeros_hbm, out_hbm, r0, cid, base, acc, idxv,
              gbuf, gsems, ssems, isems, zsem, nchunks):
    pltpu.async_copy(zeros_hbm, acc.at[pl.ds(r0, TPR)], zsem)
    for j in range(4):
        pltpu.async_copy(sd_hbm.at[base + j], idxv.at[j], isems[j])
    for b in range(2):
        pltpu.make_async_copy(sd_hbm.at[base], idxv.at[b], isems[b]).wait()
        pltpu.async_copy(tab_hbm.at[idxv.at[b, 0]], gbuf.at[b], gsems[b])
    pltpu.make_async_copy(zeros_hbm, acc.at[pl.ds(r0, TPR)], zsem).wait()
    plsc.subcore_barrier()

    def group(g, carry):
        for b in range(4):
            c = g * 4 + b
            bb = b % 2
            pb = 1 - bb
            jn = (b + 1) % 4
            jr = (b + 3) % 4
            pltpu.make_async_copy(tab_hbm.at[idxv.at[0, 0]], gbuf.at[bb],
                                  gsems[bb]).wait()
            pltpu.async_copy(gbuf.at[bb], acc.at[idxv.at[b, 1]], ssems[bb],
                             add=True)

            @pl.when(c >= 1)
            def _():
                pltpu.make_async_copy(gbuf.at[pb], acc.at[idxv.at[0, 1]],
                                      ssems[pb]).wait()

            @pl.when((c >= 1) & (c + 3 < nchunks))
            def _():
                pltpu.async_copy(sd_hbm.at[base + c + 3], idxv.at[jr],
                                 isems[jr])

            @pl.when((c >= 1) & (c + 1 < nchunks))
            def _():
                pltpu.make_async_copy(sd_hbm.at[base], idxv.at[jn],
                                      isems[jn]).wait()
                pltpu.async_copy(tab_hbm.at[idxv.at[jn, 0]], gbuf.at[pb],
                                 gsems[pb])
        return carry

    lax.fori_loop(0, nchunks // 4, group, 0)
    pltpu.make_async_copy(gbuf.at[(nchunks - 1) % 2], acc.at[idxv.at[0, 1]],
                          ssems[(nchunks - 1) % 2]).wait()
    plsc.subcore_barrier()
    pltpu.sync_copy(acc.at[pl.ds(r0, TPR)],
                    out_hbm.at[pl.ds(cid * NP + r0, TPR)])


_AGG_SCRATCH = [
    pltpu.VMEM((4, 2, 128), jnp.int32),
    pltpu.VMEM((2, 128, 128), jnp.float32),
    pltpu.SemaphoreType.DMA,
    pltpu.SemaphoreType.DMA,
    pltpu.SemaphoreType.DMA,
    pltpu.SemaphoreType.DMA,
    pltpu.SemaphoreType.DMA,
    pltpu.SemaphoreType.DMA,
    pltpu.SemaphoreType.DMA,
    pltpu.SemaphoreType.DMA,
    pltpu.SemaphoreType.DMA,
]


@functools.partial(
    pl.kernel,
    out_type=jax.ShapeDtypeStruct((2 * NP, DH), jnp.float32),
    mesh=_mesh(),
    scratch_types=[pltpu.VMEM_SHARED((NP, DH), jnp.float32)] + _AGG_SCRATCH,
)
def _agg1_kernel(tab_hbm, sd_hbm, zeros_hbm, out_hbm,
                 acc, idxv, gbuf, g0, g1, s0, s1, i0, i1, i2, i3, z0):
    c = lax.axis_index("c")
    s = lax.axis_index("s")
    _agg_ring(tab_hbm, sd_hbm, zeros_hbm, out_hbm, s * TPR, c,
              c * NCH + s * C1, acc, idxv, gbuf,
              [g0, g1], [s0, s1], [i0, i1, i2, i3], z0, C1)


@functools.partial(
    pl.kernel,
    out_type=jax.ShapeDtypeStruct((2 * NP, D2), jnp.float32),
    mesh=_mesh(),
    scratch_types=[pltpu.VMEM_SHARED((NP, D2), jnp.float32)] + _AGG_SCRATCH,
)
def _agg2_kernel(tab_hbm, sd_hbm, zeros_hbm, out_hbm,
                 acc, idxv, gbuf, g0, g1, s0, s1, i0, i1, i2, i3, z0):
    c = lax.axis_index("c")
    s = lax.axis_index("s")
    _agg_ring(tab_hbm, sd_hbm, zeros_hbm, out_hbm, s * TPR, c,
              (c * 16 + s) * C2, acc, idxv, gbuf,
              [g0, g1], [s0, s1], [i0, i1, i2, i3], z0, C2)


def _k1_body(dst_ref, x_ref, w_ref, cout_ref, out_ref, cscr):
    i = pl.program_id(0)

    @pl.when(i == 0)
    def _():
        c = _count_hi_lo(dst_ref)
        cscr[...] = c
        cout_ref[...] = c

    dinv3 = _dinv3(cscr[pl.ds(i * 16, 16)])
    y = jnp.dot(x_ref[...], w_ref[...], preferred_element_type=jnp.float32)
    y = (y.reshape(16, 128, D) * dinv3).reshape(RB, D)
    out_ref[0] = y[:, :DH]
    out_ref[1] = y[:, DH:]


def _k1(dstE, x, W1):
    return pl.pallas_call(
        _k1_body,
        grid=(NP // RB,),
        in_specs=[
            pl.BlockSpec((EP // EB, EB), lambda i: (0, 0)),
            pl.BlockSpec((RB, D), lambda i: (i, 0)),
            pl.BlockSpec((D, D), lambda i: (0, 0)),
        ],
        out_specs=[
            pl.BlockSpec((80, 128), lambda i: (0, 0)),
            pl.BlockSpec((2, RB, DH), lambda i: (0, i, 0)),
        ],
        out_shape=[
            jax.ShapeDtypeStruct((80, 128), jnp.float32),
            jax.ShapeDtypeStruct((2, NP, DH), jnp.float32),
        ],
        scratch_shapes=[pltpu.VMEM((80, 128), jnp.float32)],
    )(dstE, x, W1)


def _k2_body(c_ref, agg_ref, xs_ref, b1_ref, w2_ref, x1_ref, xw2_ref):
    i = pl.program_id(0)
    dinv3 = _dinv3(c_ref[pl.ds(i * 16, 16)])
    h = jnp.concatenate(
        [agg_ref[0] + xs_ref[0], agg_ref[1] + xs_ref[1]], axis=1)
    h = (h.reshape(16, 128, D) * dinv3).reshape(RB, D)
    x1 = jnp.maximum(h + b1_ref[...], 0.0)
    x1_ref[...] = x1
    y2 = jnp.dot(x1, w2_ref[...], preferred_element_type=jnp.float32)
    xw2_ref[...] = (y2.reshape(16, 128, D2) * dinv3).reshape(RB, D2)


def _k2(degC, aggR, xsR, b1, W2p):
    return pl.pallas_call(
        _k2_body,
        grid=(NP // RB,),
        in_specs=[
            pl.BlockSpec((80, 128), lambda i: (0, 0)),
            pl.BlockSpec((2, RB, DH), lambda i: (0, i, 0)),
            pl.BlockSpec((2, RB, DH), lambda i: (0, i, 0)),
            pl.BlockSpec((1, D), lambda i: (0, 0)),
            pl.BlockSpec((D, D2), lambda i: (0, 0)),
        ],
        out_specs=[
            pl.BlockSpec((RB, D), lambda i: (i, 0)),
            pl.BlockSpec((RB, D2), lambda i: (i, 0)),
        ],
        out_shape=[
            jax.ShapeDtypeStruct((N, D), jnp.float32),
            jax.ShapeDtypeStruct((NP, D2), jnp.float32),
        ],
    )(degC, aggR, xsR, b1, W2p)


def _k3_body(c_ref, agg_ref, xw2_ref, b2_ref, out_ref):
    i = pl.program_id(0)
    dinv3 = _dinv3(c_ref[pl.ds(i * 16, 16)])
    h = agg_ref[0] + agg_ref[1] + xw2_ref[...]
    h = (h.reshape(16, 128, D2) * dinv3).reshape(RB, D2)
    out_ref[...] = h[:, :40] + b2_ref[...]


def _k3(degC, agg2R, xw2, b2p):
    return pl.pallas_call(
        _k3_body,
        grid=(NP // RB,),
        in_specs=[
            pl.BlockSpec((80, 128), lambda i: (0, 0)),
            pl.BlockSpec((2, RB, D2), lambda i: (0, i, 0)),
            pl.BlockSpec((RB, D2), lambda i: (i, 0)),
            pl.BlockSpec((1, 40), lambda i: (0, 0)),
        ],
        out_specs=pl.BlockSpec((RB, 40), lambda i: (i, 0)),
        out_shape=jax.ShapeDtypeStruct((N, 40), jnp.float32),
    )(degC, agg2R, xw2, b2p)


@jax.jit
def _run(x, edge_index, W1, b1, W2, b2):
    ei = edge_index.astype(jnp.int32)
    pad_idx = N + (jnp.arange(EP - E, dtype=jnp.int32) % (NP - N))
    src = jnp.concatenate([ei[0], pad_idx])
    dst = jnp.concatenate([ei[1], pad_idx])
    src2d = src.reshape(NCH, 128)
    dst2d = dst.reshape(NCH, 128)
    sd = jnp.stack([src2d, dst2d], axis=1)
    sd1 = jnp.concatenate(
        [sd, jnp.stack([src2d + NP, dst2d], axis=1)], axis=0)
    xp = jnp.pad(x, ((0, NP - N), (0, 0)))
    W2p = jnp.pad(W2, ((0, 0), (0, D2 - W2.shape[1])))
    b2p = b2.reshape(1, 40)
    zeros128 = jnp.zeros((TPR, DH), jnp.float32)

    degC, xs1 = _k1(dst.reshape(EP // EB, EB), xp, W1)
    agg1 = _agg1_kernel(xs1.reshape(2 * NP, DH), sd1, zeros128)
    x1, xw2 = _k2(degC, agg1.reshape(2, NP, DH), xs1,
                  b1.reshape(1, D), W2p)
    agg2 = _agg2_kernel(xw2, sd, zeros128)
    x2 = _k3(degC, agg2.reshape(2, NP, D2), xw2, b2p)
    return x1, x2


def kernel(x, edge_index, W1, b1, W2, b2):
    return _run(x, edge_index, W1, b1, W2, b2)


# speedup vs baseline: 20.9131x; 1.1760x over previous
import functools

import jax
import jax.numpy as jnp
from jax import lax
from jax.experimental import pallas as pl
from jax.experimental.pallas import tpu as pltpu
from jax.experimental.pallas import tpu_sc as plsc

N = 10000
NP = 10016
E = 160000
EP = 163840
D = 256
DH = 128
D2 = 128
RB = 2048
TPR = 632
C1 = 80
C2 = 40
NCH = EP // 128


def _mesh():
    return plsc.VectorSubcoreMesh(core_axis_name="c", subcore_axis_name="s")


EB = 16384


def _count_hi_lo(dst_ref):
    c = jnp.zeros((80, 128), jnp.float32)
    for r in range(EP // EB):
        d = dst_ref[r]
        hi = jax.lax.shift_right_logical(d, 7)
        lo = jax.lax.bitwise_and(d, 127)
        a = (jax.lax.broadcasted_iota(jnp.int32, (80, EB), 0) == hi[None, :]
             ).astype(jnp.bfloat16)
        b = (jax.lax.broadcasted_iota(jnp.int32, (128, EB), 0) == lo[None, :]
             ).astype(jnp.bfloat16)
        c = c + jax.lax.dot_general(a, b, (((1,), (1,)), ((), ())),
                                    preferred_element_type=jnp.float32)
    return c


def _dinv3(c_blk):
    return lax.rsqrt(c_blk + 1.0)[:, :, None]


def _agg_ring(tab_hbm, sd_hbm, zeros_hbm, out_hbm, r0, cid, base, acc, idxv,
              gbuf, gsems, ssems, isems, zsem, nchunks):
    last = r0 == 15 * TPR

    @pl.when(~last)
    def _():
        pltpu.async_copy(zeros_hbm.at[pl.ds(0, TPR)], acc.at[pl.ds(r0, TPR)],
                         zsem)

    @pl.when(last)
    def _():
        pltpu.async_copy(zeros_hbm.at[pl.ds(0, 536)],
                         acc.at[pl.ds(15 * TPR, 536)], zsem)

    for j in range(4):
        pltpu.async_copy(sd_hbm.at[base + j], idxv.at[j], isems[j])
    for b in range(3):
        pltpu.make_async_copy(sd_hbm.at[base], idxv.at[b], isems[b]).wait()
        pltpu.async_copy(tab_hbm.at[idxv.at[b, 0]], gbuf.at[b], gsems[b])
    @pl.when(~last)
    def _():
        pltpu.make_async_copy(zeros_hbm.at[pl.ds(0, TPR)],
                              acc.at[pl.ds(r0, TPR)], zsem).wait()

    @pl.when(last)
    def _():
        pltpu.make_async_copy(zeros_hbm.at[pl.ds(0, 536)],
                              acc.at[pl.ds(15 * TPR, 536)], zsem).wait()

    plsc.subcore_barrier()

    def step(c, b12, dyn):
        sb = b12 % 3
        pbuf = (b12 + 2) % 3
        sj = b12 % 4
        jr = (b12 + 3) % 4
        jg = (b12 + 2) % 4
        pltpu.make_async_copy(tab_hbm.at[idxv.at[0, 0]], gbuf.at[sb],
                              gsems[sb]).wait()
        pltpu.async_copy(gbuf.at[sb], acc.at[idxv.at[sj, 1]], ssems[sb],
                         add=True)

        def wait_prev_scatter():
            pltpu.make_async_copy(gbuf.at[pbuf], acc.at[idxv.at[0, 1]],
                                  ssems[pbuf]).wait()

        def reload_idx():
            pltpu.async_copy(sd_hbm.at[base + c + 3], idxv.at[jr], isems[jr])

        def refill_gather():
            pltpu.make_async_copy(sd_hbm.at[base], idxv.at[jg],
                                  isems[jg]).wait()
            pltpu.async_copy(tab_hbm.at[idxv.at[jg, 0]], gbuf.at[pbuf],
                             gsems[pbuf])

        if dyn:
            pl.when(c >= 1)(wait_prev_scatter)
            pl.when((c >= 1) & (c + 3 < nchunks))(reload_idx)
            pl.when((c >= 1) & (c + 2 < nchunks))(refill_gather)
        else:
            if c >= 1:
                wait_prev_scatter()
            if c >= 1 and c + 3 < nchunks:
                reload_idx()
            if c >= 1 and c + 2 < nchunks:
                refill_gather()

    ngrp = nchunks // 12

    def group(g, carry):
        for b12 in range(12):
            step(g * 12 + b12, b12, True)
        return carry

    lax.fori_loop(0, ngrp, group, 0)
    for k in range(nchunks % 12):
        step(ngrp * 12 + k, k, False)
    pltpu.make_async_copy(gbuf.at[(nchunks - 1) % 3], acc.at[idxv.at[0, 1]],
                          ssems[(nchunks - 1) % 3]).wait()
    plsc.subcore_barrier()

    @pl.when(~last)
    def _():
        pltpu.sync_copy(acc.at[pl.ds(r0, TPR)],
                        out_hbm.at[pl.ds(cid * NP + r0, TPR)])

    @pl.when(last)
    def _():
        pltpu.sync_copy(acc.at[pl.ds(15 * TPR, 536)],
                        out_hbm.at[pl.ds(cid * NP + 15 * TPR, 536)])


_AGG_SCRATCH = [
    pltpu.VMEM((4, 2, 128), jnp.int32),
    pltpu.VMEM((3, 128, 128), jnp.float32),
] + [pltpu.SemaphoreType.DMA] * 11


@functools.partial(
    pl.kernel,
    out_type=jax.ShapeDtypeStruct((2 * NP, DH), jnp.float32),
    mesh=_mesh(),
    scratch_types=[pltpu.VMEM_SHARED((NP, DH), jnp.float32)] + _AGG_SCRATCH,
)
def _agg1_kernel(tab_hbm, sd_hbm, zeros_hbm, out_hbm,
                 acc, idxv, gbuf, g0, g1, g2, s0, s1, s2,
                 i0, i1, i2, i3, z0):
    c = lax.axis_index("c")
    s = lax.axis_index("s")
    _agg_ring(tab_hbm, sd_hbm, zeros_hbm, out_hbm, s * TPR, c,
              c * NCH + s * C1, acc, idxv, gbuf,
              [g0, g1, g2], [s0, s1, s2], [i0, i1, i2, i3], z0, C1)


@functools.partial(
    pl.kernel,
    out_type=jax.ShapeDtypeStruct((2 * NP, D2), jnp.float32),
    mesh=_mesh(),
    scratch_types=[pltpu.VMEM_SHARED((NP, D2), jnp.float32)] + _AGG_SCRATCH,
)
def _agg2_kernel(tab_hbm, sd_hbm, zeros_hbm, out_hbm,
                 acc, idxv, gbuf, g0, g1, g2, s0, s1, s2,
                 i0, i1, i2, i3, z0):
    c = lax.axis_index("c")
    s = lax.axis_index("s")
    _agg_ring(tab_hbm, sd_hbm, zeros_hbm, out_hbm, s * TPR, c,
              (c * 16 + s) * C2, acc, idxv, gbuf,
              [g0, g1, g2], [s0, s1, s2], [i0, i1, i2, i3], z0, C2)


def _k1_body(dst_ref, x_ref, w_ref, cout_ref, out_ref, cscr):
    i = pl.program_id(0)

    @pl.when(i == 0)
    def _():
        c = _count_hi_lo(dst_ref)
        cscr[...] = c
        cout_ref[...] = c

    dinv3 = _dinv3(cscr[pl.ds(i * 16, 16)])
    y = jnp.dot(x_ref[...], w_ref[...], preferred_element_type=jnp.float32)
    y = (y.reshape(16, 128, D) * dinv3).reshape(RB, D)
    out_ref[0] = y[:, :DH]
    out_ref[1] = y[:, DH:]


def _k1(dstE, x, W1):
    return pl.pallas_call(
        _k1_body,
        grid=(NP // RB,),
        in_specs=[
            pl.BlockSpec((EP // EB, EB), lambda i: (0, 0)),
            pl.BlockSpec((RB, D), lambda i: (i, 0)),
            pl.BlockSpec((D, D), lambda i: (0, 0)),
        ],
        out_specs=[
            pl.BlockSpec((80, 128), lambda i: (0, 0)),
            pl.BlockSpec((2, RB, DH), lambda i: (0, i, 0)),
        ],
        out_shape=[
            jax.ShapeDtypeStruct((80, 128), jnp.float32),
            jax.ShapeDtypeStruct((2, NP, DH), jnp.float32),
        ],
        scratch_shapes=[pltpu.VMEM((80, 128), jnp.float32)],
    )(dstE, x, W1)


def _k2_body(c_ref, agg_ref, xs_ref, b1_ref, w2_ref, x1_ref, xw2_ref):
    i = pl.program_id(0)
    dinv3 = _dinv3(c_ref[pl.ds(i * 16, 16)])
    h = jnp.concatenate(
        [agg_ref[0] + xs_ref[0], agg_ref[1] + xs_ref[1]], axis=1)
    h = (h.reshape(16, 128, D) * dinv3).reshape(RB, D)
    x1 = jnp.maximum(h + b1_ref[...], 0.0)
    x1_ref[...] = x1
    y2 = jnp.dot(x1, w2_ref[...], preferred_element_type=jnp.float32)
    xw2_ref[...] = (y2.reshape(16, 128, D2) * dinv3).reshape(RB, D2)


def _k2(degC, aggR, xsR, b1, W2p):
    return pl.pallas_call(
        _k2_body,
        grid=(NP // RB,),
        in_specs=[
            pl.BlockSpec((80, 128), lambda i: (0, 0)),
            pl.BlockSpec((2, RB, DH), lambda i: (0, i, 0)),
            pl.BlockSpec((2, RB, DH), lambda i: (0, i, 0)),
            pl.BlockSpec((1, D), lambda i: (0, 0)),
            pl.BlockSpec((D, D2), lambda i: (0, 0)),
        ],
        out_specs=[
            pl.BlockSpec((RB, D), lambda i: (i, 0)),
            pl.BlockSpec((RB, D2), lambda i: (i, 0)),
        ],
        out_shape=[
            jax.ShapeDtypeStruct((N, D), jnp.float32),
            jax.ShapeDtypeStruct((NP, D2), jnp.float32),
        ],
    )(degC, aggR, xsR, b1, W2p)


def _k3_body(c_ref, agg_ref, xw2_ref, b2_ref, out_ref):
    i = pl.program_id(0)
    dinv3 = _dinv3(c_ref[pl.ds(i * 16, 16)])
    h = agg_ref[0] + agg_ref[1] + xw2_ref[...]
    h = (h.reshape(16, 128, D2) * dinv3).reshape(RB, D2)
    out_ref[...] = h[:, :40] + b2_ref[...]


def _k3(degC, agg2R, xw2, b2p):
    return pl.pallas_call(
        _k3_body,
        grid=(NP // RB,),
        in_specs=[
            pl.BlockSpec((80, 128), lambda i: (0, 0)),
            pl.BlockSpec((2, RB, D2), lambda i: (0, i, 0)),
            pl.BlockSpec((RB, D2), lambda i: (i, 0)),
            pl.BlockSpec((1, 40), lambda i: (0, 0)),
        ],
        out_specs=pl.BlockSpec((RB, 40), lambda i: (i, 0)),
        out_shape=jax.ShapeDtypeStruct((N, 40), jnp.float32),
    )(degC, agg2R, xw2, b2p)


@jax.jit
def _run(x, edge_index, W1, b1, W2, b2):
    ei = edge_index.astype(jnp.int32)
    pad_idx = N + (jnp.arange(EP - E, dtype=jnp.int32) % (NP - N))
    src = jnp.concatenate([ei[0], pad_idx])
    dst = jnp.concatenate([ei[1], pad_idx])
    src2d = src.reshape(NCH, 128)
    dst2d = dst.reshape(NCH, 128)
    sd = jnp.stack([src2d, dst2d], axis=1)
    sd1 = jnp.concatenate(
        [sd, jnp.stack([src2d + NP, dst2d], axis=1)], axis=0)
    xp = jnp.pad(x, ((0, NP - N), (0, 0)))
    W2p = jnp.pad(W2, ((0, 0), (0, D2 - W2.shape[1])))
    b2p = b2.reshape(1, 40)
    zeros128 = jnp.zeros((TPR, DH), jnp.float32)

    degC, xs1 = _k1(dst.reshape(EP // EB, EB), xp, W1)
    agg1 = _agg1_kernel(xs1.reshape(2 * NP, DH), sd1, zeros128)
    x1, xw2 = _k2(degC, agg1.reshape(2, NP, DH), xs1,
                  b1.reshape(1, D), W2p)
    agg2 = _agg2_kernel(xw2, sd, zeros128)
    x2 = _k3(degC, agg2.reshape(2, NP, D2), xw2, b2p)
    return x1, x2


def kernel(x, edge_index, W1, b1, W2, b2):
    return _run(x, edge_index, W1, b1, W2, b2)
